# baseline (speedup 1.0000x reference)
import functools

import jax
import jax.numpy as jnp
import numpy as np
from jax import lax
from jax.experimental import pallas as pl
from jax.experimental.pallas import tpu as pltpu
from jax.experimental.pallas import tpu_sc as plsc

_N_BINS = 20
_BN = 2048
_BOUNDS = [float(v) for v in np.linspace(0.0, 1.0, _N_BINS + 1,
                                         dtype=np.float32)[1:_N_BINS]]

_NW = 16
_LANES = 16


def _conf_acc_body(logits_ref, labels_ref, conf_ref, acc_ref):
    x = logits_ref[...]
    c = x.shape[0]
    m = jnp.max(x, axis=0, keepdims=True)
    rows = jax.lax.broadcasted_iota(jnp.int32, x.shape, 0)
    amax = jnp.min(jnp.where(x == m, rows, c), axis=0, keepdims=True)
    s = jnp.sum(jnp.exp(x - m), axis=0, keepdims=True)
    conf_ref[...] = 1.0 / s
    acc_ref[...] = (amax == labels_ref[...]).astype(jnp.float32)


def _tc_conf_acc(logits, labels):
    n, c = logits.shape
    nsteps = n // _BN
    xt = logits.T
    labels2 = labels.astype(jnp.int32).reshape(1, n)
    return pl.pallas_call(
        _conf_acc_body,
        grid=(nsteps,),
        in_specs=[
            pl.BlockSpec((c, _BN), lambda i: (0, i)),
            pl.BlockSpec((1, _BN), lambda i: (0, i)),
        ],
        out_specs=[
            pl.BlockSpec((1, _BN), lambda i: (0, i)),
            pl.BlockSpec((1, _BN), lambda i: (0, i)),
        ],
        out_shape=[
            jax.ShapeDtypeStruct((1, n), jnp.float32),
            jax.ShapeDtypeStruct((1, n), jnp.float32),
        ],
    )(xt, labels2)


def _make_sc_hist(n):
    chunk = n // _NW
    niter = chunk // _LANES
    hist_slots = 512
    mesh = plsc.VectorSubcoreMesh(core_axis_name="c", subcore_axis_name="s",
                                  num_cores=1)

    @functools.partial(
        pl.kernel,
        mesh=mesh,
        out_type=jax.ShapeDtypeStruct((_LANES,), jnp.float32),
        compiler_params=pltpu.CompilerParams(needs_layout_passes=False),
        scratch_types=[
            pltpu.VMEM((chunk,), jnp.float32),
            pltpu.VMEM((chunk,), jnp.float32),
            pltpu.VMEM((hist_slots,), jnp.float32),
            pltpu.VMEM_SHARED((_NW, hist_slots), jnp.float32),
            pltpu.VMEM((_NW, hist_slots), jnp.float32),
            pltpu.VMEM((_LANES,), jnp.float32),
        ],
    )
    def sc_hist(conf_hbm, acc_hbm, out_hbm, conf_v, acc_v, hist_v, shared_v,
                all_v, res_v):
        wid = lax.axis_index("s")
        base = wid * chunk
        pltpu.sync_copy(conf_hbm.at[pl.ds(base, chunk)], conf_v)
        pltpu.sync_copy(acc_hbm.at[pl.ds(base, chunk)], acc_v)

        zero = jnp.zeros((_LANES,), jnp.float32)

        def body(j, carry):
            off = j * _LANES
            cv = conf_v[pl.ds(off, _LANES)]
            av = acc_v[pl.ds(off, _LANES)]
            d = cv - av
            out = [carry[0] + d]
            for k, bk in enumerate(_BOUNDS):
                m = cv > jnp.full((_LANES,), bk, jnp.float32)
                out.append(carry[k + 1] + jnp.where(m, d, zero))
            return tuple(out)

        s = lax.fori_loop(0, niter, body, tuple([zero] * _N_BINS))
        for b in range(hist_slots // _LANES):
            if b < _N_BINS:
                hi = s[b + 1] if b + 1 < _N_BINS else zero
                hist_v[pl.ds(b * _LANES, _LANES)] = s[b] - hi
            else:
                hist_v[pl.ds(b * _LANES, _LANES)] = zero

        pltpu.sync_copy(hist_v, shared_v.at[wid])
        plsc.subcore_barrier()

        @pl.when(wid == 0)
        def _():
            pltpu.sync_copy(shared_v, all_v)
            ece = jnp.float32(0.0)
            for t in range(_N_BINS):
                accum = jnp.zeros((_LANES,), jnp.float32)
                for w in range(_NW):
                    accum += all_v[w, pl.ds(t * _LANES, _LANES)]
                ece += jnp.abs(jnp.sum(accum))
            res_v[...] = jnp.full((_LANES,), ece * (1.0 / n), jnp.float32)
            pltpu.sync_copy(res_v, out_hbm)

    return sc_hist


def kernel(logits, labels):
    n, _ = logits.shape
    conf2d, acc2d = _tc_conf_acc(logits, labels)
    out16 = _make_sc_hist(n)(conf2d.reshape(n), acc2d.reshape(n))
    return out16[0:1]
